# Initial kernel scaffold; baseline (speedup 1.0000x reference)
import jax
import jax.numpy as jnp
from jax.experimental import pallas as pl


def kernel(pairs_list, pairs_num, cuda, x_single, x_cls, sentence_attn_mask, W1, b1, W2, b2):
    raise NotImplementedError("write your pallas kernel here")


# speedup vs baseline: 1.4267x; 1.4267x over previous
import functools

import jax
import jax.numpy as jnp
from jax import lax
from jax.experimental import pallas as pl
from jax.experimental.pallas import tpu as pltpu
from jax.experimental.pallas import tpu_sc as plsc

_BS, _SENT, _SEQ, _H, _NPAIRS = 8, 32, 128, 1024, 64
_BLK = 8
_NROWS = _BS * _SENT * _SENT
_ZBLK = 64
_NCORES, _NSUB = 2, 16
_ROWS_PER_TILE = _NROWS // (_NCORES * _NSUB)
_PAIRS_PER_TILE = (_BS * _NPAIRS) // (_NCORES * _NSUB)


def _attn_body(x_ref, w1_ref, b1_ref, w2_ref, m_ref, o_ref):
    x = x_ref[...]
    xm = x.reshape(_BLK * _SEQ, _H)
    t = jnp.tanh(
        jnp.dot(xm, w1_ref[...], preferred_element_type=jnp.float32)
        + b1_ref[...]
    )
    s = jnp.dot(t, w2_ref[...], preferred_element_type=jnp.float32)
    s = s.reshape(_BLK, _SEQ) + (1.0 - m_ref[...]) * (-10000.0)
    s = s - jnp.max(s, axis=1, keepdims=True)
    e = jnp.exp(s)
    p = e / jnp.sum(e, axis=1, keepdims=True)
    o_ref[...] = jnp.sum(p[:, :, None] * x, axis=1)


@functools.cache
def _sc_scatter_fn():
    mesh = plsc.VectorSubcoreMesh(core_axis_name="c", subcore_axis_name="s")

    @functools.partial(
        pl.kernel,
        mesh=mesh,
        out_type=jax.ShapeDtypeStruct((_NROWS, _H), jnp.float32),
        scratch_types=[
            pltpu.VMEM((_PAIRS_PER_TILE,), jnp.int32),
            pltpu.VMEM((_PAIRS_PER_TILE, _H), jnp.float32),
            pltpu.VMEM((_ZBLK, _H), jnp.float32),
            pltpu.SemaphoreType.DMA,
        ],
    )
    def _sc_scatter(idx_hbm, src_hbm, zeros_hbm, out_hbm, idx_v, rows_v, zeros_v, sem):
        c = lax.axis_index("c")
        s = lax.axis_index("s")
        pltpu.sync_copy(zeros_hbm, zeros_v)
        base_z = c * (_NROWS // 2) + s * _ROWS_PER_TILE
        copies = [
            pltpu.async_copy(zeros_v, out_hbm.at[pl.ds(base_z + k * _ZBLK, _ZBLK)], sem)
            for k in range(_ROWS_PER_TILE // _ZBLK)
        ]
        for cp in copies:
            cp.wait()
        plsc.subcore_barrier()
        base_p = c * (_NROWS // 2 // (_SENT * _SENT)) * _NPAIRS + s * _PAIRS_PER_TILE
        pltpu.sync_copy(idx_hbm.at[pl.ds(base_p, _PAIRS_PER_TILE)], idx_v)
        pltpu.sync_copy(src_hbm.at[pl.ds(base_p, _PAIRS_PER_TILE)], rows_v)
        pltpu.async_copy(rows_v, out_hbm.at[idx_v], sem).wait()

    return _sc_scatter


def kernel(pairs_list, pairs_num, cuda, x_single, x_cls, sentence_attn_mask, W1, b1, W2, b2):
    bs, sent, seq, hidden = x_single.shape
    npairs = pairs_list.shape[1]

    xr = x_single.reshape(bs * sent, seq, hidden)
    mr = sentence_attn_mask.reshape(bs * sent, seq)
    final = pl.pallas_call(
        _attn_body,
        grid=((bs * sent) // _BLK,),
        in_specs=[
            pl.BlockSpec((_BLK, seq, hidden), lambda i: (i, 0, 0)),
            pl.BlockSpec((hidden, hidden), lambda i: (0, 0)),
            pl.BlockSpec((1, hidden), lambda i: (0, 0)),
            pl.BlockSpec((hidden, 1), lambda i: (0, 0)),
            pl.BlockSpec((_BLK, seq), lambda i: (i, 0)),
        ],
        out_specs=pl.BlockSpec((_BLK, hidden), lambda i: (i, 0)),
        out_shape=jax.ShapeDtypeStruct((bs * sent, hidden), jnp.float32),
    )(xr, W1, b1.reshape(1, hidden), W2, mr)
    final_sent = final.reshape(bs, sent, hidden)

    pr0 = pairs_list[:, :, 0]
    pr1 = pairs_list[:, :, 1]
    ar = jnp.arange(npairs, dtype=jnp.int32)
    valid = ar[None, :] < pairs_num[:, None]
    slot = pr0 * sent + pr1
    j_gt_i = ar[None, :] > ar[:, None]
    superseded = jnp.any(
        (slot[:, :, None] == slot[:, None, :]) & j_gt_i[None] & valid[:, None, :],
        axis=2,
    )
    winner = valid & ~superseded
    cand = jnp.arange(npairs + 1, dtype=jnp.int32)
    occupied = jnp.any(
        winner[:, None, :] & (slot[:, None, :] == cand[None, :, None]), axis=2
    )
    free_slot = jnp.argmin(occupied, axis=1).astype(jnp.int32)
    dest = jnp.where(winner, slot, free_slot[:, None])
    dest = dest + (jnp.arange(bs, dtype=jnp.int32) * (sent * sent))[:, None]
    dest_flat = dest.reshape(bs * npairs)
    src = jnp.where(winner[:, :, None], x_cls[:, :, 0, :], 0.0)
    src = src.reshape(bs * npairs, hidden)
    zeros_rows = jnp.zeros((_ZBLK, hidden), jnp.float32)

    cls_mat = _sc_scatter_fn()(dest_flat, src, zeros_rows)
    cls_output_matrix_nn = cls_mat.reshape(bs, sent, sent, hidden)
    return final_sent, cls_output_matrix_nn
